# baseline (speedup 1.0000x reference)
import functools

import jax
import jax.numpy as jnp
from jax import lax
from jax.experimental import pallas as pl
from jax.experimental.pallas import tpu as pltpu
from jax.experimental.pallas import tpu_sc as plsc

N = 4096
D = 1024
E = 8
TOPK = 2
FF = 2048
NP = N * TOPK
TILE = 256
NBLK = (NP + E * TILE) // TILE
PADN = NBLK * TILE
VMEM_LIMIT = 110 * 1024 * 1024

SC_NC = 2
SC_NS = 16
SC_NW = SC_NC * SC_NS


def _silu(v):
    return v * jax.nn.sigmoid(v)


def _router_body(x_ref, r_ref, w1_ref, w2_ref, i1_ref, i2_ref):
    logits = jnp.dot(x_ref[...], r_ref[...], preferred_element_type=jnp.float32)
    m = jnp.max(logits, axis=1, keepdims=True)
    ex = jnp.exp(logits - m)
    probs = ex / jnp.sum(ex, axis=1, keepdims=True)
    idx = lax.broadcasted_iota(jnp.int32, probs.shape, 1)
    w1 = jnp.max(probs, axis=1)
    i1 = jnp.argmax(probs, axis=1).astype(jnp.int32)
    masked = jnp.where(idx == i1[:, None], -1.0, probs)
    w2 = jnp.max(masked, axis=1)
    i2 = jnp.argmax(masked, axis=1).astype(jnp.int32)
    w1_ref[...] = w1[:, None]
    w2_ref[...] = w2[:, None]
    i1_ref[...] = i1[:, None]
    i2_ref[...] = i2[:, None]


def _route(xf, router):
    TB = 1024
    return pl.pallas_call(
        _router_body,
        grid=(N // TB,),
        in_specs=[
            pl.BlockSpec((TB, D), lambda i: (i, 0)),
            pl.BlockSpec((D, E), lambda i: (0, 0)),
        ],
        out_specs=[pl.BlockSpec((TB, 1), lambda i: (i, 0))] * 4,
        out_shape=[
            jax.ShapeDtypeStruct((N, 1), jnp.float32),
            jax.ShapeDtypeStruct((N, 1), jnp.float32),
            jax.ShapeDtypeStruct((N, 1), jnp.int32),
            jax.ShapeDtypeStruct((N, 1), jnp.int32),
        ],
        compiler_params=pltpu.CompilerParams(
            dimension_semantics=("arbitrary",)),
    )(xf, router)


def _shared_body(xb_ref, sg_ref, su_ref, sd_ref, shared_ref):
    xb = xb_ref[...]
    sg = sg_ref[...].astype(jnp.bfloat16)
    su = su_ref[...].astype(jnp.bfloat16)
    sd = sd_ref[...].astype(jnp.bfloat16)
    h = _silu(jnp.dot(xb, sg, preferred_element_type=jnp.float32))
    h = h * jnp.dot(xb, su, preferred_element_type=jnp.float32)
    shared_ref[...] = jnp.dot(h.astype(jnp.bfloat16), sd,
                              preferred_element_type=jnp.float32)


def _shared_ffn(xb, sg, su, sd):
    TB = 512
    return pl.pallas_call(
        _shared_body,
        grid=(N // TB,),
        in_specs=[
            pl.BlockSpec((TB, D), lambda i: (i, 0)),
            pl.BlockSpec((D, FF), lambda i: (0, 0)),
            pl.BlockSpec((D, FF), lambda i: (0, 0)),
            pl.BlockSpec((FF, D), lambda i: (0, 0)),
        ],
        out_specs=pl.BlockSpec((TB, D), lambda i: (i, 0)),
        out_shape=jax.ShapeDtypeStruct((N, D), jnp.float32),
        compiler_params=pltpu.CompilerParams(
            dimension_semantics=("arbitrary",),
            vmem_limit_bytes=VMEM_LIMIT),
    )(xb, sg, su, sd)


def _sc_gather(table, idx, rows_total, d, chunk):
    per_w = rows_total // SC_NW
    n_chunks = per_w // chunk
    mesh = plsc.VectorSubcoreMesh(core_axis_name="c", subcore_axis_name="s")

    @functools.partial(
        pl.kernel,
        out_type=jax.ShapeDtypeStruct((rows_total, d), table.dtype),
        mesh=mesh,
        scratch_types=[
            pltpu.VMEM((chunk,), jnp.int32),
            pltpu.VMEM((chunk,), jnp.int32),
            pltpu.VMEM((chunk, d), table.dtype),
            pltpu.VMEM((chunk, d), table.dtype),
            pltpu.SemaphoreType.DMA,
            pltpu.SemaphoreType.DMA,
            pltpu.SemaphoreType.DMA,
            pltpu.SemaphoreType.DMA,
        ],
    )
    def k(table_hbm, idx_hbm, out_hbm,
          idx_v0, idx_v1, rows_v0, rows_v1, g0, g1, w0, w1):
        idx_v = (idx_v0, idx_v1)
        rows_v = (rows_v0, rows_v1)
        gsem = (g0, g1)
        wsem = (w0, w1)
        wid = lax.axis_index("c") * SC_NS + lax.axis_index("s")
        base = wid * per_w

        def off(c):
            return base + c * chunk

        pltpu.sync_copy(idx_hbm.at[pl.ds(off(0), chunk)], idx_v[0])
        pltpu.async_copy(table_hbm.at[idx_v[0]], rows_v[0], gsem[0])
        for c in range(n_chunks):
            b = c % 2
            nb = (c + 1) % 2
            if c + 1 < n_chunks:
                pltpu.sync_copy(idx_hbm.at[pl.ds(off(c + 1), chunk)],
                                idx_v[nb])
                if c >= 1:
                    pltpu.make_async_copy(
                        rows_v[nb], out_hbm.at[pl.ds(off(c - 1), chunk)],
                        wsem[nb]).wait()
                pltpu.async_copy(table_hbm.at[idx_v[nb]], rows_v[nb],
                                 gsem[nb])
            pltpu.make_async_copy(table_hbm.at[idx_v[b]], rows_v[b],
                                  gsem[b]).wait()
            pltpu.async_copy(rows_v[b], out_hbm.at[pl.ds(off(c), chunk)],
                             wsem[b])
        pltpu.make_async_copy(
            rows_v[(n_chunks - 1) % 2],
            out_hbm.at[pl.ds(off(n_chunks - 1), chunk)],
            wsem[(n_chunks - 1) % 2]).wait()
        if n_chunks >= 2:
            pltpu.make_async_copy(
                rows_v[n_chunks % 2],
                out_hbm.at[pl.ds(off(n_chunks - 2), chunk)],
                wsem[n_chunks % 2]).wait()

    return k(table, idx)


def _combine_body(sh_ref, y0_ref, y1_ref, out_ref):
    out_ref[...] = sh_ref[...] + y0_ref[...] + y1_ref[...]


def _combine(shared, yg):
    TB = 256
    half = N // TB
    return pl.pallas_call(
        _combine_body,
        grid=(half,),
        in_specs=[
            pl.BlockSpec((TB, D), lambda i: (i, 0)),
            pl.BlockSpec((TB, D), lambda i: (i, 0)),
            pl.BlockSpec((TB, D), lambda i: (i + half, 0)),
        ],
        out_specs=pl.BlockSpec((TB, D), lambda i: (i, 0)),
        out_shape=jax.ShapeDtypeStruct((N, D), jnp.float32),
        compiler_params=pltpu.CompilerParams(
            dimension_semantics=("arbitrary",)),
    )(shared, yg, yg)


def _ffn_body(eid_ref, xs_ref, w_ref, g_ref, u_ref, d_ref, ys_ref):
    xs = xs_ref[...]
    g = g_ref[0].astype(jnp.bfloat16)
    u = u_ref[0].astype(jnp.bfloat16)
    d = d_ref[0].astype(jnp.bfloat16)
    h = _silu(jnp.dot(xs, g, preferred_element_type=jnp.float32))
    h = h * jnp.dot(xs, u, preferred_element_type=jnp.float32)
    h = h * w_ref[...]
    ys_ref[...] = jnp.dot(h.astype(jnp.bfloat16), d,
                          preferred_element_type=jnp.float32)


def _expert_ffn(xs, w_s, blk_eid, gate, up, down):
    grid_spec = pltpu.PrefetchScalarGridSpec(
        num_scalar_prefetch=1,
        grid=(NBLK,),
        in_specs=[
            pl.BlockSpec((TILE, D), lambda i, e: (i, 0)),
            pl.BlockSpec((TILE, 1), lambda i, e: (i, 0)),
            pl.BlockSpec((1, D, FF), lambda i, e: (e[i], 0, 0)),
            pl.BlockSpec((1, D, FF), lambda i, e: (e[i], 0, 0)),
            pl.BlockSpec((1, FF, D), lambda i, e: (e[i], 0, 0)),
        ],
        out_specs=pl.BlockSpec((TILE, D), lambda i, e: (i, 0)),
    )
    return pl.pallas_call(
        _ffn_body,
        grid_spec=grid_spec,
        out_shape=jax.ShapeDtypeStruct((PADN, D), jnp.float32),
        compiler_params=pltpu.CompilerParams(
            dimension_semantics=("arbitrary",),
            vmem_limit_bytes=VMEM_LIMIT),
    )(blk_eid, xs, w_s, gate, up, down)


@jax.jit
def kernel(x, router, shared_gate, shared_up, shared_down, gate, up, down):
    B, T, _ = x.shape
    xf = x.reshape(N, D)
    xb = xf.astype(jnp.bfloat16)

    w1, w2, i1, i2 = _route(xf, router)

    ei = jnp.concatenate([i1[:, 0], i2[:, 0]])
    wi = jnp.concatenate([w1[:, 0], w2[:, 0]])
    onehot = (ei[:, None] == jnp.arange(E, dtype=jnp.int32)[None, :])
    csum = jnp.cumsum(onehot.astype(jnp.int32), axis=0)
    counts = csum[-1]
    rank = jnp.take_along_axis(csum, ei[:, None], axis=1)[:, 0] - 1
    pcounts = ((counts + TILE - 1) // TILE) * TILE
    pstart = jnp.concatenate(
        [jnp.zeros(1, counts.dtype), jnp.cumsum(pcounts)[:-1]])
    dest = (pstart[ei] + rank).astype(jnp.int32)
    tok = jnp.tile(jnp.arange(N, dtype=jnp.int32), 2)
    tok_s = jnp.zeros(PADN, jnp.int32).at[dest].set(tok)
    w_s = jnp.zeros((PADN, 1), jnp.float32).at[dest, 0].set(wi)
    bstart = pstart // TILE
    blk = jnp.arange(NBLK, dtype=bstart.dtype)
    blk_eid = (jnp.sum(blk[:, None] >= bstart[None, :], axis=1) - 1
               ).astype(jnp.int32)

    xb32 = lax.bitcast_convert_type(xb.reshape(N, D // 2, 2), jnp.int32)
    xs32 = _sc_gather(xb32, tok_s, PADN, D // 2, 64)
    xs = lax.bitcast_convert_type(xs32, jnp.bfloat16).reshape(PADN, D)

    shared = _shared_ffn(xb, shared_gate, shared_up, shared_down)

    ys = _expert_ffn(xs, w_s, blk_eid, gate, up, down)

    yg = _sc_gather(ys, dest, NP, D, 32)

    out = _combine(shared, yg)
    return out.reshape(B, T, D)


# speedup vs baseline: 1.0280x; 1.0280x over previous
import functools

import jax
import jax.numpy as jnp
from jax import lax
from jax.experimental import pallas as pl
from jax.experimental.pallas import tpu as pltpu
from jax.experimental.pallas import tpu_sc as plsc

N = 4096
D = 1024
E = 8
TOPK = 2
FF = 2048
NP = N * TOPK
TILE = 256
NBLK = (NP + E * TILE) // TILE
PADN = NBLK * TILE
VMEM_LIMIT = 110 * 1024 * 1024

SC_NC = 2
SC_NS = 16
SC_NW = SC_NC * SC_NS


def _silu(v):
    return v * jax.nn.sigmoid(v)


def _router_body(x_ref, r_ref, w1_ref, w2_ref, i1_ref, i2_ref):
    logits = jnp.dot(x_ref[...], r_ref[...], preferred_element_type=jnp.float32)
    m = jnp.max(logits, axis=1, keepdims=True)
    ex = jnp.exp(logits - m)
    probs = ex / jnp.sum(ex, axis=1, keepdims=True)
    idx = lax.broadcasted_iota(jnp.int32, probs.shape, 1)
    w1 = jnp.max(probs, axis=1)
    i1 = jnp.argmax(probs, axis=1).astype(jnp.int32)
    masked = jnp.where(idx == i1[:, None], -1.0, probs)
    w2 = jnp.max(masked, axis=1)
    i2 = jnp.argmax(masked, axis=1).astype(jnp.int32)
    w1_ref[...] = w1[:, None]
    w2_ref[...] = w2[:, None]
    i1_ref[...] = i1[:, None]
    i2_ref[...] = i2[:, None]


def _route(xf, router):
    TB = 1024
    return pl.pallas_call(
        _router_body,
        grid=(N // TB,),
        in_specs=[
            pl.BlockSpec((TB, D), lambda i: (i, 0)),
            pl.BlockSpec((D, E), lambda i: (0, 0)),
        ],
        out_specs=[pl.BlockSpec((TB, 1), lambda i: (i, 0))] * 4,
        out_shape=[
            jax.ShapeDtypeStruct((N, 1), jnp.float32),
            jax.ShapeDtypeStruct((N, 1), jnp.float32),
            jax.ShapeDtypeStruct((N, 1), jnp.int32),
            jax.ShapeDtypeStruct((N, 1), jnp.int32),
        ],
        compiler_params=pltpu.CompilerParams(
            dimension_semantics=("arbitrary",)),
    )(xf, router)


def _shared_body(xb_ref, sg_ref, su_ref, sd_ref, shared_ref):
    xb = xb_ref[...]
    h = _silu(jnp.dot(xb, sg_ref[...], preferred_element_type=jnp.float32))
    h = h * jnp.dot(xb, su_ref[...], preferred_element_type=jnp.float32)
    shared_ref[...] = jnp.dot(h, sd_ref[...],
                              preferred_element_type=jnp.float32)


def _shared_ffn(xb, sg, su, sd):
    TB = 512
    return pl.pallas_call(
        _shared_body,
        grid=(N // TB,),
        in_specs=[
            pl.BlockSpec((TB, D), lambda i: (i, 0)),
            pl.BlockSpec((D, FF), lambda i: (0, 0)),
            pl.BlockSpec((D, FF), lambda i: (0, 0)),
            pl.BlockSpec((FF, D), lambda i: (0, 0)),
        ],
        out_specs=pl.BlockSpec((TB, D), lambda i: (i, 0)),
        out_shape=jax.ShapeDtypeStruct((N, D), jnp.float32),
        compiler_params=pltpu.CompilerParams(
            dimension_semantics=("arbitrary",),
            vmem_limit_bytes=VMEM_LIMIT),
    )(xb, sg, su, sd)


def _sc_gather(table, idx, rows_total, d, chunk):
    per_w = rows_total // SC_NW
    n_chunks = per_w // chunk
    mesh = plsc.VectorSubcoreMesh(core_axis_name="c", subcore_axis_name="s")

    @functools.partial(
        pl.kernel,
        out_type=jax.ShapeDtypeStruct((rows_total, d), table.dtype),
        mesh=mesh,
        scratch_types=[
            pltpu.VMEM((chunk,), jnp.int32),
            pltpu.VMEM((chunk,), jnp.int32),
            pltpu.VMEM((chunk, d), table.dtype),
            pltpu.VMEM((chunk, d), table.dtype),
            pltpu.SemaphoreType.DMA,
            pltpu.SemaphoreType.DMA,
            pltpu.SemaphoreType.DMA,
            pltpu.SemaphoreType.DMA,
        ],
    )
    def k(table_hbm, idx_hbm, out_hbm,
          idx_v0, idx_v1, rows_v0, rows_v1, g0, g1, w0, w1):
        idx_v = (idx_v0, idx_v1)
        rows_v = (rows_v0, rows_v1)
        gsem = (g0, g1)
        wsem = (w0, w1)
        wid = lax.axis_index("c") * SC_NS + lax.axis_index("s")
        base = wid * per_w

        def off(c):
            return base + c * chunk

        pltpu.sync_copy(idx_hbm.at[pl.ds(off(0), chunk)], idx_v[0])
        pltpu.async_copy(table_hbm.at[idx_v[0]], rows_v[0], gsem[0])
        for c in range(n_chunks):
            b = c % 2
            nb = (c + 1) % 2
            if c + 1 < n_chunks:
                pltpu.sync_copy(idx_hbm.at[pl.ds(off(c + 1), chunk)],
                                idx_v[nb])
                if c >= 1:
                    pltpu.make_async_copy(
                        rows_v[nb], out_hbm.at[pl.ds(off(c - 1), chunk)],
                        wsem[nb]).wait()
                pltpu.async_copy(table_hbm.at[idx_v[nb]], rows_v[nb],
                                 gsem[nb])
            pltpu.make_async_copy(table_hbm.at[idx_v[b]], rows_v[b],
                                  gsem[b]).wait()
            pltpu.async_copy(rows_v[b], out_hbm.at[pl.ds(off(c), chunk)],
                             wsem[b])
        pltpu.make_async_copy(
            rows_v[(n_chunks - 1) % 2],
            out_hbm.at[pl.ds(off(n_chunks - 1), chunk)],
            wsem[(n_chunks - 1) % 2]).wait()
        if n_chunks >= 2:
            pltpu.make_async_copy(
                rows_v[n_chunks % 2],
                out_hbm.at[pl.ds(off(n_chunks - 2), chunk)],
                wsem[n_chunks % 2]).wait()

    return k(table, idx)


def _combine_body(sh_ref, y0_ref, y1_ref, out_ref):
    out_ref[...] = sh_ref[...] + y0_ref[...] + y1_ref[...]


def _combine(shared, yg):
    TB = 256
    half = N // TB
    return pl.pallas_call(
        _combine_body,
        grid=(half,),
        in_specs=[
            pl.BlockSpec((TB, D), lambda i: (i, 0)),
            pl.BlockSpec((TB, D), lambda i: (i, 0)),
            pl.BlockSpec((TB, D), lambda i: (i + half, 0)),
        ],
        out_specs=pl.BlockSpec((TB, D), lambda i: (i, 0)),
        out_shape=jax.ShapeDtypeStruct((N, D), jnp.float32),
        compiler_params=pltpu.CompilerParams(
            dimension_semantics=("arbitrary",)),
    )(shared, yg, yg)


def _ffn_body(eid_ref, xs_ref, w_ref, g_ref, u_ref, d_ref, ys_ref):
    xs = xs_ref[...].astype(jnp.float32)
    h = _silu(jnp.dot(xs, g_ref[0], preferred_element_type=jnp.float32))
    h = h * jnp.dot(xs, u_ref[0], preferred_element_type=jnp.float32)
    h = h * w_ref[...]
    ys_ref[...] = jnp.dot(h, d_ref[0], preferred_element_type=jnp.float32)


def _expert_ffn(xs, w_s, blk_eid, gate, up, down):
    grid_spec = pltpu.PrefetchScalarGridSpec(
        num_scalar_prefetch=1,
        grid=(NBLK,),
        in_specs=[
            pl.BlockSpec((TILE, D), lambda i, e: (i, 0)),
            pl.BlockSpec((TILE, 1), lambda i, e: (i, 0)),
            pl.BlockSpec((1, D, FF), lambda i, e: (e[i], 0, 0)),
            pl.BlockSpec((1, D, FF), lambda i, e: (e[i], 0, 0)),
            pl.BlockSpec((1, FF, D), lambda i, e: (e[i], 0, 0)),
        ],
        out_specs=pl.BlockSpec((TILE, D), lambda i, e: (i, 0)),
    )
    return pl.pallas_call(
        _ffn_body,
        grid_spec=grid_spec,
        out_shape=jax.ShapeDtypeStruct((PADN, D), jnp.float32),
        compiler_params=pltpu.CompilerParams(
            dimension_semantics=("arbitrary",),
            vmem_limit_bytes=VMEM_LIMIT),
    )(blk_eid, xs, w_s, gate, up, down)


@jax.jit
def kernel(x, router, shared_gate, shared_up, shared_down, gate, up, down):
    B, T, _ = x.shape
    xf = x.reshape(N, D)
    xb = xf.astype(jnp.bfloat16)

    w1, w2, i1, i2 = _route(xf, router)

    ei = jnp.concatenate([i1[:, 0], i2[:, 0]])
    wi = jnp.concatenate([w1[:, 0], w2[:, 0]])
    onehot = (ei[:, None] == jnp.arange(E, dtype=jnp.int32)[None, :])
    csum = jnp.cumsum(onehot.astype(jnp.int32), axis=0)
    counts = csum[-1]
    rank = jnp.take_along_axis(csum, ei[:, None], axis=1)[:, 0] - 1
    pcounts = ((counts + TILE - 1) // TILE) * TILE
    pstart = jnp.concatenate(
        [jnp.zeros(1, counts.dtype), jnp.cumsum(pcounts)[:-1]])
    dest = (pstart[ei] + rank).astype(jnp.int32)
    tok = jnp.tile(jnp.arange(N, dtype=jnp.int32), 2)
    tok_s = jnp.zeros(PADN, jnp.int32).at[dest].set(tok)
    w_s = jnp.zeros((PADN, 1), jnp.float32).at[dest, 0].set(wi)
    bstart = pstart // TILE
    blk = jnp.arange(NBLK, dtype=bstart.dtype)
    blk_eid = (jnp.sum(blk[:, None] >= bstart[None, :], axis=1) - 1
               ).astype(jnp.int32)

    xb32 = lax.bitcast_convert_type(xb.reshape(N, D // 2, 2), jnp.int32)
    xs32 = _sc_gather(xb32, tok_s, PADN, D // 2, 64)
    xs = lax.bitcast_convert_type(xs32, jnp.bfloat16).reshape(PADN, D)

    shared = _shared_ffn(xf, shared_gate, shared_up, shared_down)

    ys = _expert_ffn(xs, w_s, blk_eid, gate, up, down)

    yg = _sc_gather(ys, dest, NP, D, 32)

    out = _combine(shared, yg)
    return out.reshape(B, T, D)


# speedup vs baseline: 1.0633x; 1.0343x over previous
import functools

import jax
import jax.numpy as jnp
from jax import lax
from jax.experimental import pallas as pl
from jax.experimental.pallas import tpu as pltpu
from jax.experimental.pallas import tpu_sc as plsc

N = 4096
D = 1024
E = 8
TOPK = 2
FF = 2048
NP = N * TOPK
TILE = 256
NBLK = (NP + E * TILE) // TILE
PADN = NBLK * TILE
VMEM_LIMIT = 110 * 1024 * 1024

SC_NC = 2
SC_NS = 16
SC_NW = SC_NC * SC_NS


def _silu(v):
    return v * jax.nn.sigmoid(v)


def _router_body(x_ref, r_ref, w1_ref, w2_ref, i1_ref, i2_ref):
    logits = jnp.dot(x_ref[...], r_ref[...], preferred_element_type=jnp.float32)
    m = jnp.max(logits, axis=1, keepdims=True)
    ex = jnp.exp(logits - m)
    probs = ex / jnp.sum(ex, axis=1, keepdims=True)
    idx = lax.broadcasted_iota(jnp.int32, probs.shape, 1)
    w1 = jnp.max(probs, axis=1)
    i1 = jnp.argmax(probs, axis=1).astype(jnp.int32)
    masked = jnp.where(idx == i1[:, None], -1.0, probs)
    w2 = jnp.max(masked, axis=1)
    i2 = jnp.argmax(masked, axis=1).astype(jnp.int32)
    w1_ref[...] = w1[:, None]
    w2_ref[...] = w2[:, None]
    i1_ref[...] = i1[:, None]
    i2_ref[...] = i2[:, None]


def _route(xf, router):
    TB = 1024
    return pl.pallas_call(
        _router_body,
        grid=(N // TB,),
        in_specs=[
            pl.BlockSpec((TB, D), lambda i: (i, 0)),
            pl.BlockSpec((D, E), lambda i: (0, 0)),
        ],
        out_specs=[pl.BlockSpec((TB, 1), lambda i: (i, 0))] * 4,
        out_shape=[
            jax.ShapeDtypeStruct((N, 1), jnp.float32),
            jax.ShapeDtypeStruct((N, 1), jnp.float32),
            jax.ShapeDtypeStruct((N, 1), jnp.int32),
            jax.ShapeDtypeStruct((N, 1), jnp.int32),
        ],
        compiler_params=pltpu.CompilerParams(
            dimension_semantics=("arbitrary",)),
    )(xf, router)


def _shared_body(xb_ref, sg_ref, su_ref, sd_ref, shared_ref):
    xb = xb_ref[...]
    h = _silu(jnp.dot(xb, sg_ref[...], preferred_element_type=jnp.float32))
    h = h * jnp.dot(xb, su_ref[...], preferred_element_type=jnp.float32)
    shared_ref[...] = jnp.dot(h.astype(jnp.bfloat16), sd_ref[...],
                              preferred_element_type=jnp.float32)


def _shared_ffn(xb, sg, su, sd):
    TB = 512
    return pl.pallas_call(
        _shared_body,
        grid=(N // TB,),
        in_specs=[
            pl.BlockSpec((TB, D), lambda i: (i, 0)),
            pl.BlockSpec((D, FF), lambda i: (0, 0)),
            pl.BlockSpec((D, FF), lambda i: (0, 0)),
            pl.BlockSpec((FF, D), lambda i: (0, 0)),
        ],
        out_specs=pl.BlockSpec((TB, D), lambda i: (i, 0)),
        out_shape=jax.ShapeDtypeStruct((N, D), jnp.float32),
        compiler_params=pltpu.CompilerParams(
            dimension_semantics=("arbitrary",),
            vmem_limit_bytes=VMEM_LIMIT),
    )(xb, sg, su, sd)


def _sc_gather(table, idx, rows_total, d, chunk):
    per_w = rows_total // SC_NW
    n_chunks = per_w // chunk
    mesh = plsc.VectorSubcoreMesh(core_axis_name="c", subcore_axis_name="s")

    @functools.partial(
        pl.kernel,
        out_type=jax.ShapeDtypeStruct((rows_total, d), table.dtype),
        mesh=mesh,
        scratch_types=[
            pltpu.VMEM((chunk,), jnp.int32),
            pltpu.VMEM((chunk,), jnp.int32),
            pltpu.VMEM((chunk, d), table.dtype),
            pltpu.VMEM((chunk, d), table.dtype),
            pltpu.SemaphoreType.DMA,
            pltpu.SemaphoreType.DMA,
            pltpu.SemaphoreType.DMA,
            pltpu.SemaphoreType.DMA,
        ],
    )
    def k(table_hbm, idx_hbm, out_hbm,
          idx_v0, idx_v1, rows_v0, rows_v1, g0, g1, w0, w1):
        idx_v = (idx_v0, idx_v1)
        rows_v = (rows_v0, rows_v1)
        gsem = (g0, g1)
        wsem = (w0, w1)
        wid = lax.axis_index("c") * SC_NS + lax.axis_index("s")
        base = wid * per_w

        def off(c):
            return base + c * chunk

        pltpu.sync_copy(idx_hbm.at[pl.ds(off(0), chunk)], idx_v[0])
        pltpu.async_copy(table_hbm.at[idx_v[0]], rows_v[0], gsem[0])
        for c in range(n_chunks):
            b = c % 2
            nb = (c + 1) % 2
            if c + 1 < n_chunks:
                pltpu.sync_copy(idx_hbm.at[pl.ds(off(c + 1), chunk)],
                                idx_v[nb])
                if c >= 1:
                    pltpu.make_async_copy(
                        rows_v[nb], out_hbm.at[pl.ds(off(c - 1), chunk)],
                        wsem[nb]).wait()
                pltpu.async_copy(table_hbm.at[idx_v[nb]], rows_v[nb],
                                 gsem[nb])
            pltpu.make_async_copy(table_hbm.at[idx_v[b]], rows_v[b],
                                  gsem[b]).wait()
            pltpu.async_copy(rows_v[b], out_hbm.at[pl.ds(off(c), chunk)],
                             wsem[b])
        pltpu.make_async_copy(
            rows_v[(n_chunks - 1) % 2],
            out_hbm.at[pl.ds(off(n_chunks - 1), chunk)],
            wsem[(n_chunks - 1) % 2]).wait()
        if n_chunks >= 2:
            pltpu.make_async_copy(
                rows_v[n_chunks % 2],
                out_hbm.at[pl.ds(off(n_chunks - 2), chunk)],
                wsem[n_chunks % 2]).wait()

    return k(table, idx)


def _sc_dispatch(xb32, dest, wi):
    d2 = D // 2
    per_w = NP // SC_NW
    chunk = 64
    n_chunks = per_w // chunk
    mesh = plsc.VectorSubcoreMesh(core_axis_name="c", subcore_axis_name="s")

    @functools.partial(
        pl.kernel,
        out_type=(
            jax.ShapeDtypeStruct((PADN, d2), jnp.int32),
            jax.ShapeDtypeStruct((PADN,), jnp.float32),
        ),
        mesh=mesh,
        scratch_types=[
            pltpu.VMEM((chunk,), jnp.int32),
            pltpu.VMEM((chunk,), jnp.int32),
            pltpu.VMEM((chunk, d2), jnp.int32),
            pltpu.VMEM((chunk, d2), jnp.int32),
            pltpu.VMEM((chunk,), jnp.float32),
            pltpu.VMEM((chunk,), jnp.float32),
            pltpu.SemaphoreType.DMA,
            pltpu.SemaphoreType.DMA,
            pltpu.SemaphoreType.DMA,
            pltpu.SemaphoreType.DMA,
        ],
    )
    def k(x_hbm, dest_hbm, wi_hbm, xs_hbm, ws_hbm,
          dv0, dv1, rv0, rv1, wv0, wv1, rs0, rs1, ws0, ws1):
        dv = (dv0, dv1)
        rv = (rv0, rv1)
        wv = (wv0, wv1)
        rsem = (rs0, rs1)
        wsem = (ws0, ws1)
        wid = lax.axis_index("c") * SC_NS + lax.axis_index("s")
        base = wid * per_w
        src_base = base - jnp.where(base >= N, N, 0)
        for c in range(n_chunks):
            b = c % 2
            off = base + c * chunk
            soff = src_base + c * chunk
            if c >= 2:
                pltpu.make_async_copy(rv[b], xs_hbm.at[dv[b]],
                                      rsem[b]).wait()
                pltpu.make_async_copy(wv[b], ws_hbm.at[dv[b]],
                                      wsem[b]).wait()
            pltpu.sync_copy(dest_hbm.at[pl.ds(off, chunk)], dv[b])
            pltpu.sync_copy(x_hbm.at[pl.ds(soff, chunk)], rv[b])
            pltpu.sync_copy(wi_hbm.at[pl.ds(off, chunk)], wv[b])
            pltpu.async_copy(rv[b], xs_hbm.at[dv[b]], rsem[b])
            pltpu.async_copy(wv[b], ws_hbm.at[dv[b]], wsem[b])
        for c in range(max(0, n_chunks - 2), n_chunks):
            b = c % 2
            pltpu.make_async_copy(rv[b], xs_hbm.at[dv[b]], rsem[b]).wait()
            pltpu.make_async_copy(wv[b], ws_hbm.at[dv[b]], wsem[b]).wait()

    return k(xb32, dest, wi)


def _combine_body(sh_ref, y0_ref, y1_ref, out_ref):
    out_ref[...] = sh_ref[...] + y0_ref[...] + y1_ref[...]


def _combine(shared, yg):
    TB = 256
    half = N // TB
    return pl.pallas_call(
        _combine_body,
        grid=(half,),
        in_specs=[
            pl.BlockSpec((TB, D), lambda i: (i, 0)),
            pl.BlockSpec((TB, D), lambda i: (i, 0)),
            pl.BlockSpec((TB, D), lambda i: (i + half, 0)),
        ],
        out_specs=pl.BlockSpec((TB, D), lambda i: (i, 0)),
        out_shape=jax.ShapeDtypeStruct((N, D), jnp.float32),
        compiler_params=pltpu.CompilerParams(
            dimension_semantics=("arbitrary",)),
    )(shared, yg, yg)


def _hi_body(eid_ref, xs_ref, w_ref, g_ref, u_ref, h_ref,
             gb_ref, ub_ref, last_ref):
    i = pl.program_id(0)
    e = eid_ref[i]
    changed = jnp.logical_or(i == 0, e != last_ref[0])

    @pl.when(changed)
    def _():
        gb_ref[...] = g_ref[0].astype(jnp.bfloat16)
        ub_ref[...] = u_ref[0].astype(jnp.bfloat16)
        last_ref[0] = e

    xs = xs_ref[...]
    h = _silu(jnp.dot(xs, gb_ref[...], preferred_element_type=jnp.float32))
    h = h * jnp.dot(xs, ub_ref[...], preferred_element_type=jnp.float32)
    h = h * w_ref[...]
    h_ref[...] = h.astype(jnp.bfloat16)


def _down_body(eid_ref, h_ref, d_ref, ys_ref, db_ref, last_ref):
    i = pl.program_id(0)
    e = eid_ref[i]
    changed = jnp.logical_or(i == 0, e != last_ref[0])

    @pl.when(changed)
    def _():
        db_ref[...] = d_ref[0].astype(jnp.bfloat16)
        last_ref[0] = e

    ys_ref[...] = jnp.dot(h_ref[...], db_ref[...],
                          preferred_element_type=jnp.float32)


def _expert_ffn(xs, w_s, blk_eid, gate, up, down):
    hi_spec = pltpu.PrefetchScalarGridSpec(
        num_scalar_prefetch=1,
        grid=(NBLK,),
        in_specs=[
            pl.BlockSpec((TILE, D), lambda i, e: (i, 0)),
            pl.BlockSpec((TILE, 1), lambda i, e: (i, 0)),
            pl.BlockSpec((1, D, FF), lambda i, e: (e[i], 0, 0)),
            pl.BlockSpec((1, D, FF), lambda i, e: (e[i], 0, 0)),
        ],
        out_specs=pl.BlockSpec((TILE, FF), lambda i, e: (i, 0)),
        scratch_shapes=[
            pltpu.VMEM((D, FF), jnp.bfloat16),
            pltpu.VMEM((D, FF), jnp.bfloat16),
            pltpu.SMEM((1,), jnp.int32),
        ],
    )
    h = pl.pallas_call(
        _hi_body,
        grid_spec=hi_spec,
        out_shape=jax.ShapeDtypeStruct((PADN, FF), jnp.bfloat16),
        compiler_params=pltpu.CompilerParams(
            dimension_semantics=("arbitrary",),
            vmem_limit_bytes=VMEM_LIMIT),
    )(blk_eid, xs, w_s, gate, up)

    down_spec = pltpu.PrefetchScalarGridSpec(
        num_scalar_prefetch=1,
        grid=(NBLK,),
        in_specs=[
            pl.BlockSpec((TILE, FF), lambda i, e: (i, 0)),
            pl.BlockSpec((1, FF, D), lambda i, e: (e[i], 0, 0)),
        ],
        out_specs=pl.BlockSpec((TILE, D), lambda i, e: (i, 0)),
        scratch_shapes=[
            pltpu.VMEM((FF, D), jnp.bfloat16),
            pltpu.SMEM((1,), jnp.int32),
        ],
    )
    return pl.pallas_call(
        _down_body,
        grid_spec=down_spec,
        out_shape=jax.ShapeDtypeStruct((PADN, D), jnp.float32),
        compiler_params=pltpu.CompilerParams(
            dimension_semantics=("arbitrary",),
            vmem_limit_bytes=VMEM_LIMIT),
    )(blk_eid, h, down)


@jax.jit
def kernel(x, router, shared_gate, shared_up, shared_down, gate, up, down):
    B, T, _ = x.shape
    xf = x.reshape(N, D)
    xb = xf.astype(jnp.bfloat16)

    w1, w2, i1, i2 = _route(xf, router)

    ei = jnp.concatenate([i1[:, 0], i2[:, 0]])
    wi = jnp.concatenate([w1[:, 0], w2[:, 0]])
    onehot = (ei[:, None] == jnp.arange(E, dtype=jnp.int32)[None, :])
    csum = jnp.cumsum(onehot.astype(jnp.int32), axis=0)
    counts = csum[-1]
    rank = jnp.take_along_axis(csum, ei[:, None], axis=1)[:, 0] - 1
    pcounts = ((counts + TILE - 1) // TILE) * TILE
    pstart = jnp.concatenate(
        [jnp.zeros(1, counts.dtype), jnp.cumsum(pcounts)[:-1]])
    dest = (pstart[ei] + rank).astype(jnp.int32)
    bstart = pstart // TILE
    blk = jnp.arange(NBLK, dtype=bstart.dtype)
    blk_eid = (jnp.sum(blk[:, None] >= bstart[None, :], axis=1) - 1
               ).astype(jnp.int32)

    xb32 = lax.bitcast_convert_type(xb.reshape(N, D // 2, 2), jnp.int32)
    xs32, ws = _sc_dispatch(xb32, dest, wi)
    xs = lax.bitcast_convert_type(xs32, jnp.bfloat16).reshape(PADN, D)

    shared = _shared_ffn(xb,
                         shared_gate.astype(jnp.bfloat16),
                         shared_up.astype(jnp.bfloat16),
                         shared_down.astype(jnp.bfloat16))

    ys = _expert_ffn(xs, ws.reshape(PADN, 1), blk_eid,
                     gate, up, down)

    yg = _sc_gather(ys, dest, NP, D, 32)

    out = _combine(shared, yg)
    return out.reshape(B, T, D)
